# baseline (speedup 1.0000x reference)
import functools

import jax
import jax.numpy as jnp
from jax import lax
from jax.experimental import pallas as pl
from jax.experimental.pallas import tpu as pltpu
from jax.experimental.pallas import tpu_sc as plsc

N = 10000
M = 320000
D = 128
DE = 16
SLOPE = 0.2
INV_BN = (1.0 + 1e-5) ** -0.5

NTILE = 32
N_PAD = 10240
TPN = N_PAD // NTILE
M_PAD = 327680
TM = M_PAD // NTILE
CH = 128
NCH = TM // CH

SCAN_R = 32
SCAN_E = SCAN_R * 128
NSCAN = M_PAD // SCAN_E
CAP = SCAN_E + CH
ACC_R = TPN + 8


def _mesh():
    return plsc.VectorSubcoreMesh(core_axis_name="c", subcore_axis_name="s")


@functools.partial(
    pl.kernel,
    out_type=jax.ShapeDtypeStruct((2, N_PAD, D), jnp.float32),
    mesh=_mesh(),
    scratch_types=[
        pltpu.VMEM((NCH, CH), jnp.int32),
        pltpu.VMEM((NCH, CH), jnp.int32),
        pltpu.VMEM((CH, D), jnp.float32),
        pltpu.VMEM_SHARED((N_PAD, D), jnp.float32),
        pltpu.SemaphoreType.DMA,
    ],
    compiler_params=pltpu.CompilerParams(needs_layout_passes=False),
)
def _seg_sum(table, srcb, dstb, out, src_v, dst_v, rows_v, acc, sem):
    c = lax.axis_index("c")
    s = lax.axis_index("s")
    gid = c * 16 + s
    rows_per_sub = N_PAD // 16

    zv = jnp.zeros((16,), jnp.float32)

    def zrow(i, _):
        for j in range(D // 16):
            rows_v[i, pl.ds(j * 16, 16)] = zv
        return 0

    lax.fori_loop(0, CH, zrow, 0)
    for r in range(rows_per_sub // CH):
        pltpu.sync_copy(rows_v, acc.at[pl.ds(s * rows_per_sub + r * CH, CH)])
    plsc.subcore_barrier()

    pltpu.sync_copy(srcb.at[gid], src_v)
    pltpu.sync_copy(dstb.at[gid], dst_v)

    def chunk(k, _):
        pltpu.async_copy(table.at[src_v.at[k]], rows_v, sem).wait()
        pltpu.sync_copy(rows_v, acc.at[dst_v.at[k]], add=True)
        return 0

    lax.fori_loop(0, NCH, chunk, 0)
    plsc.subcore_barrier()

    for r in range(rows_per_sub // CH):
        base = s * rows_per_sub + r * CH
        pltpu.sync_copy(acc.at[pl.ds(base, CH)], rows_v)
        pltpu.sync_copy(rows_v, out.at[c, pl.ds(base, CH)])


@functools.partial(
    pl.kernel,
    out_type=[
        jax.ShapeDtypeStruct((N_PAD, D), jnp.float32),
        jax.ShapeDtypeStruct((N_PAD,), jnp.float32),
    ],
    mesh=_mesh(),
    scratch_types=[
        pltpu.VMEM((SCAN_R, 128), jnp.int32),
        pltpu.VMEM((SCAN_R, 128), jnp.int32),
        pltpu.VMEM((CAP,), jnp.int32),
        pltpu.VMEM((CAP,), jnp.int32),
        pltpu.VMEM((CH, D), jnp.float32),
        pltpu.VMEM((ACC_R, D), jnp.float32),
        pltpu.VMEM((TPN + 16,), jnp.float32),
        pltpu.SemaphoreType.DMA,
    ],
    compiler_params=pltpu.CompilerParams(needs_layout_passes=False),
)
def _seg_max(s1t, dst2d, src2d, out, out_cnt, dscan, sscan, cb_d, cb_s, rows,
             acc, cnt_acc, sem):
    c = lax.axis_index("c")
    s = lax.axis_index("s")
    gid = c * 16 + s
    lo = gid * TPN

    zf = jnp.zeros((16,), jnp.float32)
    zi = jnp.zeros((16,), jnp.int32)
    onef = jnp.ones((16,), jnp.float32)
    padv = jnp.full((16,), TPN, jnp.int32)

    def zacc(i, _):
        for j in range(D // 16):
            acc[i, pl.ds(j * 16, 16)] = zf
        return 0

    lax.fori_loop(0, ACC_R, zacc, 0)

    def zcnt(i, _):
        cnt_acc[pl.ds(i * 16, 16)] = zf
        return 0

    lax.fori_loop(0, (TPN + 16) // 16, zcnt, 0)

    def fill(i, _):
        cb_d[pl.ds(i * 16, 16)] = padv
        cb_s[pl.ds(i * 16, 16)] = zi
        return 0

    lax.fori_loop(0, CAP // 16, fill, 0)

    def scan_chunk(b, _):
        pltpu.sync_copy(dst2d.at[pl.ds(b * SCAN_R, SCAN_R)], dscan)
        pltpu.sync_copy(src2d.at[pl.ds(b * SCAN_R, SCAN_R)], sscan)

        lov = jnp.full((16,), lo, jnp.int32)
        hiv = jnp.full((16,), lo + TPN, jnp.int32)

        def row(r, cnt):
            for q in range(8):
                dv = dscan[r, pl.ds(q * 16, 16)]
                sv = sscan[r, pl.ds(q * 16, 16)]
                m = (dv >= lov) & (dv < hiv)
                dloc = dv - lov
                inc = m.astype(jnp.int32)
                cntv = jnp.full((16,), cnt, jnp.int32)
                pos = cntv + plsc.cumsum(inc) - jnp.full((16,), 1, jnp.int32)
                plsc.store_scatter(cb_d, [pos], dloc, mask=m)
                plsc.store_scatter(cb_s, [pos], sv, mask=m)
                plsc.addupdate_scatter(cnt_acc, [dloc], onef, mask=m)
                cnt = cnt + jnp.sum(inc)
            return cnt

        cnt = lax.fori_loop(0, SCAN_R, row, jnp.int32(0))
        nblk = (cnt + CH - 1) // CH

        def blk(t, _):
            pltpu.async_copy(s1t.at[cb_s.at[pl.ds(t * CH, CH)]], rows, sem).wait()

            def edge(e, _):
                dloc = cb_d[pl.ds(t * CH + e, 16)][0]
                for j in range(D // 16):
                    sl = pl.ds(j * 16, 16)
                    acc[dloc, sl] = jnp.maximum(acc[dloc, sl], rows[e, sl])
                return 0

            lax.fori_loop(0, CH, edge, 0)
            return 0

        lax.fori_loop(0, nblk, blk, 0)

        def refill(i, _):
            cb_d[pl.ds(i * 16, 16)] = padv
            cb_s[pl.ds(i * 16, 16)] = zi
            return 0

        lax.fori_loop(0, (nblk * CH) // 16, refill, 0)
        return 0

    lax.fori_loop(0, NSCAN, scan_chunk, 0)
    pltpu.sync_copy(acc.at[pl.ds(0, TPN)], out.at[pl.ds(lo, TPN)])
    pltpu.sync_copy(cnt_acc.at[pl.ds(0, TPN)], out_cnt.at[pl.ds(lo, TPN)])


@functools.partial(
    pl.kernel,
    out_type=jax.ShapeDtypeStruct((M_PAD // 8, 128), jnp.float32),
    mesh=_mesh(),
    scratch_types=[
        pltpu.VMEM((NCH, CH), jnp.int32),
        pltpu.VMEM((NCH, CH), jnp.int32),
        pltpu.VMEM((CH, 128), jnp.float32),
        pltpu.VMEM((CH, 128), jnp.float32),
        pltpu.VMEM((CH // 8, 128), jnp.float32),
        pltpu.SemaphoreType.DMA,
        pltpu.SemaphoreType.DMA,
    ],
    compiler_params=pltpu.CompilerParams(needs_layout_passes=False),
)
def _edge_gather(combo, srcb, dstb, out, src_v, dst_v, bufa, bufc, outb, sem1, sem2):
    c = lax.axis_index("c")
    s = lax.axis_index("s")
    gid = c * 16 + s
    pltpu.sync_copy(srcb.at[gid], src_v)
    pltpu.sync_copy(dstb.at[gid], dst_v)
    obase = gid * (TM // 8)

    def chunk(k, _):
        cp1 = pltpu.async_copy(combo.at[src_v.at[k]], bufa, sem1)
        cp2 = pltpu.async_copy(combo.at[dst_v.at[k]], bufc, sem2)
        cp1.wait()
        cp2.wait()

        def edge(e, _):
            r = e // 8
            col = (e % 8) * 16
            outb[r, pl.ds(col, 16)] = (
                bufa[e, pl.ds(0, 16)] + bufc[e, pl.ds(16, 16)])
            return 0

        lax.fori_loop(0, CH, edge, 0)
        pltpu.sync_copy(outb, out.at[pl.ds(obase + k * (CH // 8), CH // 8)])
        return 0

    lax.fori_loop(0, NCH, chunk, 0)


def _relu(x):
    return jnp.maximum(x, 0.0)


def _leaky(x):
    return jnp.where(x >= 0, x, SLOPE * x)


BR = 512
GRID_N = N_PAD // BR


def _tca_body(p_ref, w1_ref, b1_ref, s1_ref):
    p = p_ref[...]
    s0 = p[0] + p[1]
    s1_ref[...] = _relu(jnp.dot(s0, w1_ref[...],
                                preferred_element_type=jnp.float32) + b1_ref[...])


def _tc_a(partials, w1, b1):
    return pl.pallas_call(
        _tca_body,
        grid=(GRID_N,),
        in_specs=[
            pl.BlockSpec((2, BR, D), lambda i: (0, i, 0)),
            pl.BlockSpec((D, D), lambda i: (0, 0)),
            pl.BlockSpec((D,), lambda i: (0,)),
        ],
        out_specs=pl.BlockSpec((BR, D), lambda i: (i, 0)),
        out_shape=jax.ShapeDtypeStruct((N_PAD, D), jnp.float32),
    )(partials, w1, b1)


def _tcb_body(p_ref, cnt_ref, smax_ref, w2_ref, b2_ref, w3_ref, b3_ref, s2_ref):
    p = p_ref[...]
    s0 = p[0] + p[1]
    cnt = cnt_ref[...].reshape(BR, 1)
    mean = s0 / jnp.maximum(cnt, 1.0)
    s2_ref[...] = _relu(
        jnp.dot(mean, w2_ref[...], preferred_element_type=jnp.float32)
        + b2_ref[...]
    ) + _relu(
        jnp.dot(smax_ref[...], w3_ref[...], preferred_element_type=jnp.float32)
        + b3_ref[...])


def _tc_b(partials, cnt, smax, w2, b2, w3, b3):
    return pl.pallas_call(
        _tcb_body,
        grid=(GRID_N,),
        in_specs=[
            pl.BlockSpec((2, BR, D), lambda i: (0, i, 0)),
            pl.BlockSpec((BR,), lambda i: (i,)),
            pl.BlockSpec((BR, D), lambda i: (i, 0)),
            pl.BlockSpec((D, D), lambda i: (0, 0)),
            pl.BlockSpec((D,), lambda i: (0,)),
            pl.BlockSpec((D, D), lambda i: (0, 0)),
            pl.BlockSpec((D,), lambda i: (0,)),
        ],
        out_specs=pl.BlockSpec((BR, D), lambda i: (i, 0)),
        out_shape=jax.ShapeDtypeStruct((N_PAD, D), jnp.float32),
    )(partials, cnt, smax, w2, b2, w3, b3)


def _tcc_body(p_ref, s1_ref, s2_ref, v_ref, w5_ref, b5_ref, wc_ref, bc_ref,
              ws_ref, vout_ref, combo_ref):
    p = p_ref[...]
    ss2 = p[0] + p[1]
    s1 = s1_ref[...]
    s2 = s2_ref[...]
    s3 = s1 + _relu(jnp.dot(ss2, w5_ref[...],
                            preferred_element_type=jnp.float32) + b5_ref[...])
    wc = wc_ref[...]
    vc = (jnp.dot(s1, wc[:D], preferred_element_type=jnp.float32)
          + jnp.dot(s2, wc[D:2 * D], preferred_element_type=jnp.float32)
          + jnp.dot(s3, wc[2 * D:], preferred_element_type=jnp.float32)
          + bc_ref[...])
    vout_ref[...] = _leaky(vc * INV_BN) + v_ref[...]
    ws = ws_ref[...]
    a = jnp.dot(vc, ws[:D], preferred_element_type=jnp.float32)
    ce = jnp.dot(vc, ws[D + DE:], preferred_element_type=jnp.float32)
    combo_ref[...] = jnp.concatenate(
        [a, ce, jnp.zeros((BR, 128 - 2 * DE), jnp.float32)], axis=1)


def _tc_c(partials2, s1, s2, vpad, w5, b5, wc, bc, ws):
    return pl.pallas_call(
        _tcc_body,
        grid=(GRID_N,),
        in_specs=[
            pl.BlockSpec((2, BR, D), lambda i: (0, i, 0)),
            pl.BlockSpec((BR, D), lambda i: (i, 0)),
            pl.BlockSpec((BR, D), lambda i: (i, 0)),
            pl.BlockSpec((BR, D), lambda i: (i, 0)),
            pl.BlockSpec((D, D), lambda i: (0, 0)),
            pl.BlockSpec((D,), lambda i: (0,)),
            pl.BlockSpec((3 * D, D), lambda i: (0, 0)),
            pl.BlockSpec((D,), lambda i: (0,)),
            pl.BlockSpec((2 * D + DE, DE), lambda i: (0, 0)),
        ],
        out_specs=[
            pl.BlockSpec((BR, D), lambda i: (i, 0)),
            pl.BlockSpec((BR, 128), lambda i: (i, 0)),
        ],
        out_shape=[
            jax.ShapeDtypeStruct((N_PAD, D), jnp.float32),
            jax.ShapeDtypeStruct((N_PAD, 128), jnp.float32),
        ],
    )(partials2, s1, s2, vpad, w5, b5, wc, bc, ws)


ER = 800
GRID_E = (M // 8) // ER


def _tce_body(er_ref, g1_ref, wd_ref, bsr_ref, out_ref):
    er = er_ref[...]
    t = jnp.dot(_leaky(er), wd_ref[...],
                preferred_element_type=jnp.float32) + bsr_ref[...]
    out_ref[...] = _leaky((g1_ref[...] + t) * INV_BN) + er


def _tc_e(er, g1r, wd, bsr):
    return pl.pallas_call(
        _tce_body,
        grid=(GRID_E,),
        in_specs=[
            pl.BlockSpec((ER, 128), lambda i: (i, 0)),
            pl.BlockSpec((ER, 128), lambda i: (i, 0)),
            pl.BlockSpec((128, 128), lambda i: (0, 0)),
            pl.BlockSpec((128,), lambda i: (0,)),
        ],
        out_specs=pl.BlockSpec((ER, 128), lambda i: (i, 0)),
        out_shape=jax.ShapeDtypeStruct((M // 8, 128), jnp.float32),
    )(er, g1r, wd, bsr)


def kernel(V, E, edge_index, Wp1, bp1, Wp2, bp2, Wp3, bp3, Wp5, bp5, Wc, bc, WS, bS):
    f32 = jnp.float32
    src = edge_index[0]
    dst = edge_index[1]
    padi = jnp.full((M_PAD - M,), N_PAD - 1, jnp.int32)
    src_p = jnp.concatenate([src, padi]).reshape(NTILE, NCH, CH)
    dst_p = jnp.concatenate([dst, padi]).reshape(NTILE, NCH, CH)
    src2d = src_p.reshape(M_PAD // 128, 128)
    dst2d = dst_p.reshape(M_PAD // 128, 128)

    vpad = jnp.zeros((N_PAD, D), f32).at[:N].set(V)

    partials = _seg_sum(vpad, src_p, dst_p)
    s1 = _tc_a(partials, Wp1, bp1)

    smax, cnt = _seg_max(s1, dst2d, src2d)
    s2 = _tc_b(partials, cnt, smax, Wp2, bp2, Wp3, bp3)

    partials2 = _seg_sum(s2, src_p, dst_p)
    vout_pad, combo = _tc_c(partials2, s1, s2, vpad, Wp5, bp5, Wc, bc, WS)

    g1 = _edge_gather(combo, src_p, dst_p)

    er = E.reshape(M // 8, 128)
    g1r = g1[:M // 8]
    wd = jnp.zeros((128, 128), f32)
    wmid = WS[D:D + DE]
    for k in range(8):
        wd = wd.at[k * DE:(k + 1) * DE, k * DE:(k + 1) * DE].set(wmid)
    bsr = jnp.tile(bS, 8)

    eoutr = _tc_e(er, g1r, wd, bsr)

    return (vout_pad[:N], eoutr.reshape(M, DE))
